# baseline (device time: 14494 ns/iter reference)
import jax
import jax.numpy as jnp
from jax import lax
from jax.experimental import pallas as pl
from jax.experimental.pallas import tpu as pltpu

N_DEV = 8
E_PER = 2


def kernel(x, router_W, route_idx, expert_W):
    n, d = x.shape
    h = expert_W.shape[-1]
    rows = n // N_DEV

    def body(x_ref, rw_ref, idx_ref, w_ref, out_ref,
             send_ref, ag_send_ref, rs_recv_ref, ag_recv_ref,
             rs_send_sems, rs_recv_sems, ag_send_sems, ag_recv_sems):
        my = lax.axis_index("i")

        barrier_sem = pltpu.get_barrier_semaphore()
        for g in range(1, N_DEV):
            pl.semaphore_signal(barrier_sem, inc=1, device_id=(my ^ g,),
                                device_id_type=pl.DeviceIdType.MESH)

        idx = idx_ref[:, :]
        xv = x_ref[:, :]
        acc = jnp.zeros((n, h), jnp.float32)
        for e in range(E_PER):
            xm = jnp.where(idx == my * E_PER + e, xv, 0.0).astype(
                jnp.bfloat16)
            acc = acc + jnp.dot(
                xm, w_ref[e].astype(jnp.bfloat16),
                preferred_element_type=jnp.float32,
            )
        send_ref[:, :] = acc.astype(jnp.bfloat16)

        pl.semaphore_wait(barrier_sem, N_DEV - 1)

        rs = []
        for g in range(1, N_DEV):
            peer = my ^ g
            rdma = pltpu.make_async_remote_copy(
                src_ref=send_ref.at[pl.ds(peer * rows, rows), :],
                dst_ref=rs_recv_ref.at[g - 1],
                send_sem=rs_send_sems.at[g - 1],
                recv_sem=rs_recv_sems.at[g - 1],
                device_id=(peer,),
                device_id_type=pl.DeviceIdType.MESH,
            )
            rdma.start()
            rs.append(rdma)

        red = send_ref[pl.ds(my * rows, rows), :].astype(jnp.float32)
        for g in range(1, N_DEV):
            rs[g - 1].wait()
            red = red + rs_recv_ref[g - 1].astype(jnp.float32)
        out_ref[pl.ds(my * rows, rows), :] = red
        ag_send_ref[:, :] = red.astype(jnp.bfloat16)

        ag = []
        for g in range(1, N_DEV):
            rdma = pltpu.make_async_remote_copy(
                src_ref=ag_send_ref,
                dst_ref=ag_recv_ref.at[g - 1],
                send_sem=ag_send_sems.at[g - 1],
                recv_sem=ag_recv_sems.at[g - 1],
                device_id=(my ^ g,),
                device_id_type=pl.DeviceIdType.MESH,
            )
            rdma.start()
            ag.append(rdma)
        for g in range(1, N_DEV):
            ag[g - 1].wait()
            out_ref[pl.ds((my ^ g) * rows, rows), :] = (
                ag_recv_ref[g - 1].astype(jnp.float32))

    return pl.pallas_call(
        body,
        out_shape=jax.ShapeDtypeStruct((n, h), jnp.float32),
        in_specs=[
            pl.BlockSpec(memory_space=pltpu.VMEM),
            pl.BlockSpec(memory_space=pltpu.VMEM),
            pl.BlockSpec(memory_space=pltpu.VMEM),
            pl.BlockSpec(memory_space=pltpu.VMEM),
        ],
        out_specs=pl.BlockSpec(memory_space=pltpu.VMEM),
        scratch_shapes=[
            pltpu.VMEM((n, h), jnp.bfloat16),
            pltpu.VMEM((rows, h), jnp.bfloat16),
            pltpu.VMEM((N_DEV - 1, rows, h), jnp.bfloat16),
            pltpu.VMEM((N_DEV - 1, rows, h), jnp.bfloat16),
            pltpu.SemaphoreType.DMA((N_DEV - 1,)),
            pltpu.SemaphoreType.DMA((N_DEV - 1,)),
            pltpu.SemaphoreType.DMA((N_DEV - 1,)),
            pltpu.SemaphoreType.DMA((N_DEV - 1,)),
        ],
        compiler_params=pltpu.CompilerParams(collective_id=0),
    )(x, router_W, route_idx, expert_W)
